# baseline (speedup 1.0000x reference)
import functools

import jax
import jax.numpy as jnp
from jax import lax
from jax.experimental import pallas as pl
from jax.experimental.pallas import tpu as pltpu
from jax.experimental.pallas import tpu_sc as plsc

NUM_DOMAINS = 4
EPS = 1e-3
B, D = 16384, 128
L = 16
NC, NS = 2, 16
NW = NC * NS
RPW = B // NW
CH = 4
CR = RPW // CH
ACC = 9 * D
BR = 256

_mesh = plsc.VectorSubcoreMesh(core_axis_name="c", subcore_axis_name="s")
_params = pltpu.CompilerParams(needs_layout_passes=False)


def _iota16():
    return lax.broadcasted_iota(jnp.int32, (L,), 0)


@functools.partial(
    pl.kernel,
    out_type=(
        jax.ShapeDtypeStruct((NW * ACC,), jnp.float32),
        jax.ShapeDtypeStruct((B,), jnp.int32),
    ),
    mesh=_mesh,
    compiler_params=_params,
    scratch_types=[
        pltpu.VMEM((RPW * D,), jnp.float32),
        pltpu.VMEM((RPW * NUM_DOMAINS,), jnp.float32),
        pltpu.VMEM((RPW,), jnp.int32),
        pltpu.VMEM((ACC,), jnp.float32),
        pltpu.SemaphoreType.DMA,
        pltpu.SemaphoreType.DMA,
        pltpu.SemaphoreType.DMA,
        pltpu.SemaphoreType.DMA,
    ],
)
def _stats_kernel(x_hbm, di_hbm, parts_hbm, didx_hbm,
                  data_v, di_v, didx_v, acc_v, s0, s1, s2, s3):
    wid = lax.axis_index("s") * NC + lax.axis_index("c")
    row0 = wid * RPW
    iota = _iota16()
    sems = [s0, s1, s2, s3]

    pltpu.sync_copy(di_hbm.at[pl.ds(row0 * NUM_DOMAINS, RPW * NUM_DOMAINS)],
                    di_v)
    cps = []
    for c in range(CH):
        cps.append(pltpu.async_copy(
            x_hbm.at[pl.ds((row0 + c * CR) * D, CR * D)],
            data_v.at[pl.ds(c * CR * D, CR * D)],
            sems[c]))

    def zero_body(q, _):
        acc_v[pl.ds(q * L, L)] = jnp.zeros((L,), jnp.float32)
        return 0

    lax.fori_loop(0, ACC // L, zero_body, 0)

    ones = jnp.ones((L,), jnp.float32)

    def didx_body(t, _):
        idx0 = t * (L * NUM_DOMAINS) + iota * NUM_DOMAINS
        best = plsc.load_gather(di_v, [idx0])
        bidx = jnp.zeros((L,), jnp.int32)
        for c in range(1, NUM_DOMAINS):
            v = plsc.load_gather(di_v, [idx0 + c])
            take = v > best
            best = jnp.where(take, v, best)
            bidx = jnp.where(take, jnp.full((L,), c, jnp.int32), bidx)
        didx_v[pl.ds(t * L, L)] = bidx
        plsc.addupdate_scatter(acc_v, [8 * D + bidx * L + iota], ones)
        return 0

    lax.fori_loop(0, RPW // L, didx_body, 0, unroll=2)

    def acc_body(r, _):
        d_b = plsc.load_gather(didx_v, [jnp.zeros((L,), jnp.int32) + r])
        col = d_b * D + iota
        vs = [data_v[pl.ds(r * D + j * L, L)] for j in range(D // L)]
        sqs = [v * v for v in vs]
        for j in range(D // L):
            plsc.addupdate_scatter(acc_v, [col + j * L], vs[j])
        for j in range(D // L):
            plsc.addupdate_scatter(acc_v, [col + j * L + 4 * D], sqs[j])
        return 0

    for c in range(CH):
        cps[c].wait()
        lax.fori_loop(c * CR, (c + 1) * CR, acc_body, 0, unroll=4)

    pltpu.sync_copy(acc_v, parts_hbm.at[pl.ds(wid * ACC, ACC)])
    pltpu.sync_copy(didx_v, didx_hbm.at[pl.ds(row0, RPW)])


def _tc_norm_body(parts_ref, didx_ref, g_ref, b_ref, x_ref, out_ref, so_ref):
    @pl.when(pl.program_id(0) == 0)
    def _():
        tot = jnp.sum(parts_ref[...].reshape(NW, 9, D), axis=0)
        for d in range(NUM_DOMAINS):
            cnt = jnp.sum(tot[8:9, d * L:(d + 1) * L])
            safe = jnp.maximum(cnt, 1.0)
            mean = tot[d:d + 1, :] / safe
            var = jnp.maximum(tot[d + 4:d + 5, :] / safe - mean * mean, 0.0)
            s = g_ref[d:d + 1, :] * lax.rsqrt(var + EPS)
            so_ref[d:d + 1, :] = s
            so_ref[d + 4:d + 5, :] = b_ref[d:d + 1, :] - mean * s

    didx = didx_ref[...]
    s_rows = jnp.zeros((BR, D), jnp.float32)
    o_rows = jnp.zeros((BR, D), jnp.float32)
    for d in range(NUM_DOMAINS):
        m = (didx == d).astype(jnp.float32)
        s_rows = s_rows + m * so_ref[d:d + 1, :]
        o_rows = o_rows + m * so_ref[d + NUM_DOMAINS:d + NUM_DOMAINS + 1, :]
    out_ref[...] = x_ref[...] * s_rows + o_rows


_tc_norm = pl.pallas_call(
    _tc_norm_body,
    grid=(B // BR,),
    in_specs=[
        pl.BlockSpec((NW, ACC), lambda i: (0, 0)),
        pl.BlockSpec((BR, 1), lambda i: (i, 0)),
        pl.BlockSpec((NUM_DOMAINS, D), lambda i: (0, 0)),
        pl.BlockSpec((NUM_DOMAINS, D), lambda i: (0, 0)),
        pl.BlockSpec((BR, D), lambda i: (i, 0)),
    ],
    out_specs=pl.BlockSpec((BR, D), lambda i: (i, 0)),
    out_shape=jax.ShapeDtypeStruct((B, D), jnp.float32),
    scratch_shapes=[pltpu.VMEM((2 * NUM_DOMAINS, D), jnp.float32)],
)


def kernel(inputs, domain_indicator, gamma, beta):
    x = inputs.reshape(-1)
    di = domain_indicator.reshape(-1)
    parts, didx = _stats_kernel(x, di)
    out = _tc_norm(parts.reshape(NW, ACC), didx.reshape(B, 1),
                   gamma[:, :D], beta[:, :D], inputs)
    return out


# speedup vs baseline: 1.3019x; 1.3019x over previous
import functools

import jax
import jax.numpy as jnp
from jax import lax
from jax.experimental import pallas as pl
from jax.experimental.pallas import tpu as pltpu
from jax.experimental.pallas import tpu_sc as plsc

NUM_DOMAINS = 4
EPS = 1e-3
B, D = 16384, 128
L = 16
NC, NS = 2, 16
JG = D // L
SR = B // NS
HR = SR // NC
CR = 128
NCH = SR // CR
RCH = HR // CR
ACC_R = 16

_mesh = plsc.VectorSubcoreMesh(core_axis_name="c", subcore_axis_name="s")
_params = pltpu.CompilerParams(needs_layout_passes=False)


def _iota16():
    return lax.broadcasted_iota(jnp.int32, (L,), 0)


def _rsqrt(x):
    i = plsc.bitcast(x, jnp.int32)
    y = plsc.bitcast(jnp.int32(0x5F3759DF) - (i >> 1), jnp.float32)
    for _ in range(3):
        y = y * (1.5 - 0.5 * x * y * y)
    return y


@functools.partial(
    pl.kernel,
    out_type=jax.ShapeDtypeStruct((B * D,), jnp.float32),
    mesh=_mesh,
    compiler_params=_params,
    scratch_types=[
        pltpu.VMEM((RCH * CR * D,), jnp.float32),
        pltpu.VMEM((2 * CR * D,), jnp.float32),
        pltpu.VMEM((SR * NUM_DOMAINS,), jnp.float32),
        pltpu.VMEM((SR,), jnp.int32),
        pltpu.VMEM((ACC_R, D), jnp.float32),
        pltpu.VMEM((ACC_R, D), jnp.float32),
        pltpu.VMEM((2 * NUM_DOMAINS * D,), jnp.float32),
        pltpu.VMEM((NUM_DOMAINS, D), jnp.float32),
        pltpu.VMEM((NUM_DOMAINS, D), jnp.float32),
        pltpu.VMEM_SHARED((ACC_R, D), jnp.float32),
    ] + [pltpu.SemaphoreType.DMA] * 10,
)
def _fused_kernel(x_hbm, di_hbm, g_hbm, b_hbm, out_hbm,
                  nbuf_v, rbuf_v, di_v, didx_v, acc_v, tot_v, so_v, g_v, b_v,
                  tot_sh, *sems):
    cid = lax.axis_index("c")
    tid = lax.axis_index("s")
    row0 = tid * SR
    noff = cid * HR
    iota = _iota16()
    rsems = list(sems[:RCH])
    tsems = list(sems[RCH:RCH + 2])
    osems = list(sems[RCH + 2:])

    pltpu.sync_copy(di_hbm.at[pl.ds(row0 * NUM_DOMAINS, SR * NUM_DOMAINS)],
                    di_v)
    pltpu.sync_copy(g_hbm.at[pl.ds(0, NUM_DOMAINS), pl.ds(0, D)], g_v)
    pltpu.sync_copy(b_hbm.at[pl.ds(0, NUM_DOMAINS), pl.ds(0, D)], b_v)

    ncps = []
    for c in range(RCH):
        ncps.append(pltpu.async_copy(
            x_hbm.at[pl.ds((row0 + noff + c * CR) * D, CR * D)],
            nbuf_v.at[pl.ds(c * CR * D, CR * D)],
            rsems[c]))

    ooff = HR - noff

    def rot_in(c):
        return pltpu.async_copy(
            x_hbm.at[pl.ds((row0 + ooff + c * CR) * D, CR * D)],
            rbuf_v.at[pl.ds((c % 2) * CR * D, CR * D)],
            tsems[c % 2])

    rcps = [rot_in(0), rot_in(1)]

    zero = jnp.zeros((L,), jnp.float32)
    for q in range(ACC_R):
        for j in range(JG):
            acc_v[q, pl.ds(j * L, L)] = zero

    @pl.when(tid == 0)
    def _():
        pltpu.sync_copy(acc_v, tot_sh)

    plsc.subcore_barrier()

    ones = jnp.ones((L,), jnp.float32)
    eights = jnp.full((L,), 8, jnp.int32)

    def didx_body(t, _):
        idx0 = t * (L * NUM_DOMAINS) + iota * NUM_DOMAINS
        best = plsc.load_gather(di_v, [idx0])
        bidx = jnp.zeros((L,), jnp.int32)
        for c in range(1, NUM_DOMAINS):
            v = plsc.load_gather(di_v, [idx0 + c])
            take = v > best
            best = jnp.where(take, v, best)
            bidx = jnp.where(take, jnp.full((L,), c, jnp.int32), bidx)
        didx_v[pl.ds(t * L, L)] = bidx
        plsc.addupdate_scatter(acc_v, [eights, bidx * L + iota], ones)
        return 0

    lax.fori_loop(0, SR // L, didx_body, 0, unroll=2)

    def make_acc_body(buf, base_buf, base_row):
        def acc_body(i, _):
            d_b = plsc.load_gather(
                didx_v, [jnp.zeros((L,), jnp.int32) + (i + base_row)])
            col = d_b * D + iota
            vs = [buf[pl.ds(base_buf + i * D + j * L, L)] for j in range(JG)]
            sqs = [v * v for v in vs]
            for j in range(JG):
                plsc.addupdate_scatter(acc_v, [d_b, j * L + iota], vs[j])
            for j in range(JG):
                plsc.addupdate_scatter(acc_v, [d_b + 4, j * L + iota], sqs[j])
            return 0
        return acc_body

    for c in range(RCH):
        ncps[c].wait()
        lax.fori_loop(
            0, CR, make_acc_body(nbuf_v, c * CR * D, noff + c * CR), 0,
            unroll=4)
    for c in range(NCH - RCH):
        rcps[c].wait()
        lax.fori_loop(
            0, CR, make_acc_body(rbuf_v, (c % 2) * CR * D, ooff + c * CR), 0,
            unroll=4)
        if c + 2 < NCH - RCH:
            rcps.append(rot_in(c + 2))

    pltpu.sync_copy(acc_v, tot_sh.at[iota], add=True)
    plsc.subcore_barrier()
    pltpu.sync_copy(tot_sh, tot_v)

    for d in range(NUM_DOMAINS):
        cnt = jnp.sum(tot_v[8, pl.ds(d * L, L)])
        safe = jnp.maximum(cnt, 1.0)
        for j in range(JG):
            off = d * D + j * L
            sm = tot_v[d, pl.ds(j * L, L)]
            sq = tot_v[d + 4, pl.ds(j * L, L)]
            mean = sm / safe
            var = jnp.maximum(sq / safe - mean * mean, 0.0)
            s = g_v[d, pl.ds(j * L, L)] * _rsqrt(var + EPS)
            so_v[pl.ds(off, L)] = s
            so_v[pl.ds(NUM_DOMAINS * D + off, L)] = \
                b_v[d, pl.ds(j * L, L)] - mean * s

    def make_norm_body(c):
        def norm_body(i, _):
            d_b = plsc.load_gather(
                didx_v, [jnp.zeros((L,), jnp.int32) + (i + noff + c * CR)])
            col = d_b * D + iota
            base = c * CR * D + i * D
            vs = [nbuf_v[pl.ds(base + j * L, L)] for j in range(JG)]
            ss = [plsc.load_gather(so_v, [col + j * L]) for j in range(JG)]
            os_ = [plsc.load_gather(
                so_v, [col + j * L + NUM_DOMAINS * D]) for j in range(JG)]
            for j in range(JG):
                nbuf_v[pl.ds(base + j * L, L)] = vs[j] * ss[j] + os_[j]
            return 0
        return norm_body

    ocps = []
    for c in range(RCH):
        lax.fori_loop(0, CR, make_norm_body(c), 0, unroll=4)
        ocps.append(pltpu.async_copy(
            nbuf_v.at[pl.ds(c * CR * D, CR * D)],
            out_hbm.at[pl.ds((row0 + noff + c * CR) * D, CR * D)],
            osems[c]))
    for c in range(RCH):
        ocps[c].wait()


def kernel(inputs, domain_indicator, gamma, beta):
    x = inputs.reshape(-1)
    di = domain_indicator.reshape(-1)
    out = _fused_kernel(x, di, gamma, beta)
    return out.reshape(B, D)


# speedup vs baseline: 1.3289x; 1.0207x over previous
import functools

import jax
import jax.numpy as jnp
from jax import lax
from jax.experimental import pallas as pl
from jax.experimental.pallas import tpu as pltpu
from jax.experimental.pallas import tpu_sc as plsc

NUM_DOMAINS = 4
EPS = 1e-3
B, D = 16384, 128
L = 16
NC, NS = 2, 16
NW = NC * NS
RPW = B // NW
CH = 4
CR = RPW // CH
ACC = 9 * D

_mesh = plsc.VectorSubcoreMesh(core_axis_name="c", subcore_axis_name="s")
_params = pltpu.CompilerParams(needs_layout_passes=False)


def _iota16():
    return lax.broadcasted_iota(jnp.int32, (L,), 0)


def _rsqrt(x):
    i = plsc.bitcast(x, jnp.int32)
    y = plsc.bitcast(jnp.int32(0x5F3759DF) - (i >> 1), jnp.float32)
    for _ in range(3):
        y = y * (1.5 - 0.5 * x * y * y)
    return y


@functools.partial(
    pl.kernel,
    out_type=(
        jax.ShapeDtypeStruct((NW * ACC,), jnp.float32),
        jax.ShapeDtypeStruct((B,), jnp.int32),
    ),
    mesh=_mesh,
    compiler_params=_params,
    scratch_types=[
        pltpu.VMEM((RPW * D,), jnp.float32),
        pltpu.VMEM((RPW * NUM_DOMAINS,), jnp.float32),
        pltpu.VMEM((RPW,), jnp.int32),
        pltpu.VMEM((ACC,), jnp.float32),
        pltpu.SemaphoreType.DMA,
        pltpu.SemaphoreType.DMA,
        pltpu.SemaphoreType.DMA,
        pltpu.SemaphoreType.DMA,
    ],
)
def _stats_kernel(x_hbm, di_hbm, parts_hbm, didx_hbm,
                  data_v, di_v, didx_v, acc_v, s0, s1, s2, s3):
    wid = lax.axis_index("s") * NC + lax.axis_index("c")
    row0 = wid * RPW
    iota = _iota16()
    sems = [s0, s1, s2, s3]

    pltpu.sync_copy(di_hbm.at[pl.ds(row0 * NUM_DOMAINS, RPW * NUM_DOMAINS)],
                    di_v)
    cps = []
    for c in range(CH):
        cps.append(pltpu.async_copy(
            x_hbm.at[pl.ds((row0 + c * CR) * D, CR * D)],
            data_v.at[pl.ds(c * CR * D, CR * D)],
            sems[c]))

    def zero_body(q, _):
        acc_v[pl.ds(q * L, L)] = jnp.zeros((L,), jnp.float32)
        return 0

    lax.fori_loop(0, ACC // L, zero_body, 0)

    ones = jnp.ones((L,), jnp.float32)

    def didx_body(t, _):
        idx0 = t * (L * NUM_DOMAINS) + iota * NUM_DOMAINS
        best = plsc.load_gather(di_v, [idx0])
        bidx = jnp.zeros((L,), jnp.int32)
        for c in range(1, NUM_DOMAINS):
            v = plsc.load_gather(di_v, [idx0 + c])
            take = v > best
            best = jnp.where(take, v, best)
            bidx = jnp.where(take, jnp.full((L,), c, jnp.int32), bidx)
        didx_v[pl.ds(t * L, L)] = bidx
        plsc.addupdate_scatter(acc_v, [8 * D + bidx * L + iota], ones)
        return 0

    lax.fori_loop(0, RPW // L, didx_body, 0, unroll=2)

    def acc_body(r, _):
        d_b = plsc.load_gather(didx_v, [jnp.zeros((L,), jnp.int32) + r])
        col = d_b * D + iota
        vs = [data_v[pl.ds(r * D + j * L, L)] for j in range(D // L)]
        sqs = [v * v for v in vs]
        for j in range(D // L):
            plsc.addupdate_scatter(acc_v, [col + j * L], vs[j])
        for j in range(D // L):
            plsc.addupdate_scatter(acc_v, [col + j * L + 4 * D], sqs[j])
        return 0

    for c in range(CH):
        cps[c].wait()
        lax.fori_loop(c * CR, (c + 1) * CR, acc_body, 0, unroll=4)

    pltpu.sync_copy(acc_v, parts_hbm.at[pl.ds(wid * ACC, ACC)])
    pltpu.sync_copy(didx_v, didx_hbm.at[pl.ds(row0, RPW)])


@functools.partial(
    pl.kernel,
    out_type=jax.ShapeDtypeStruct((B * D,), jnp.float32),
    mesh=_mesh,
    compiler_params=_params,
    scratch_types=[
        pltpu.VMEM((RPW * D,), jnp.float32),
        pltpu.VMEM((RPW,), jnp.int32),
        pltpu.VMEM((NW * ACC,), jnp.float32),
        pltpu.VMEM((ACC,), jnp.float32),
        pltpu.VMEM((8 * D,), jnp.float32),
        pltpu.VMEM((4 * D,), jnp.float32),
        pltpu.VMEM((4 * D,), jnp.float32),
        pltpu.VMEM_SHARED((8 * D,), jnp.float32),
        pltpu.SemaphoreType.DMA,
        pltpu.SemaphoreType.DMA,
        pltpu.SemaphoreType.DMA,
        pltpu.SemaphoreType.DMA,
        pltpu.SemaphoreType.DMA,
        pltpu.SemaphoreType.DMA,
        pltpu.SemaphoreType.DMA,
        pltpu.SemaphoreType.DMA,
    ],
)
def _norm_kernel(x_hbm, didx_hbm, parts_hbm, g_hbm, b_hbm, out_hbm,
                 data_v, didx_v, parts_v, tot_v, so_v, g_v, b_v, so_sh,
                 i0, i1, i2, i3, o0, o1, o2, o3):
    sid = lax.axis_index("s")
    wid = sid * NC + lax.axis_index("c")
    row0 = wid * RPW
    iota = _iota16()
    isems = [i0, i1, i2, i3]
    osems = [o0, o1, o2, o3]

    pltpu.sync_copy(didx_hbm.at[pl.ds(row0, RPW)], didx_v)
    cps = []
    for c in range(CH):
        cps.append(pltpu.async_copy(
            x_hbm.at[pl.ds((row0 + c * CR) * D, CR * D)],
            data_v.at[pl.ds(c * CR * D, CR * D)],
            isems[c]))

    @pl.when(sid == 0)
    def _():
        pltpu.sync_copy(parts_hbm, parts_v)
        pltpu.sync_copy(g_hbm, g_v)
        pltpu.sync_copy(b_hbm, b_v)

        def red_body(q, _):
            acc = [parts_v[pl.ds(w * ACC + q * L, L)] for w in range(NW)]
            while len(acc) > 1:
                acc = [a + b for a, b in zip(acc[::2], acc[1::2])]
            tot_v[pl.ds(q * L, L)] = acc[0]
            return 0

        lax.fori_loop(0, ACC // L, red_body, 0, unroll=2)

        for d in range(NUM_DOMAINS):
            cnt = jnp.sum(tot_v[pl.ds(8 * D + d * L, L)])
            safe = jnp.maximum(cnt, 1.0)
            for j in range(D // L):
                off = d * D + j * L
                sm = tot_v[pl.ds(off, L)]
                sq = tot_v[pl.ds(4 * D + off, L)]
                mean = sm / safe
                var = jnp.maximum(sq / safe - mean * mean, 0.0)
                s = g_v[pl.ds(off, L)] * _rsqrt(var + EPS)
                so_v[pl.ds(off, L)] = s
                so_v[pl.ds(4 * D + off, L)] = b_v[pl.ds(off, L)] - mean * s

        pltpu.sync_copy(so_v, so_sh)

    plsc.subcore_barrier()
    pltpu.sync_copy(so_sh, so_v)

    def norm_body(r, _):
        d_b = plsc.load_gather(didx_v, [jnp.zeros((L,), jnp.int32) + r])
        col = d_b * D + iota
        vs = [data_v[pl.ds(r * D + j * L, L)] for j in range(D // L)]
        ss = [plsc.load_gather(so_v, [col + j * L]) for j in range(D // L)]
        os_ = [plsc.load_gather(so_v, [col + j * L + 4 * D])
               for j in range(D // L)]
        for j in range(D // L):
            data_v[pl.ds(r * D + j * L, L)] = vs[j] * ss[j] + os_[j]
        return 0

    ocps = []
    for c in range(CH):
        cps[c].wait()
        lax.fori_loop(c * CR, (c + 1) * CR, norm_body, 0, unroll=4)
        ocps.append(pltpu.async_copy(
            data_v.at[pl.ds(c * CR * D, CR * D)],
            out_hbm.at[pl.ds((row0 + c * CR) * D, CR * D)],
            osems[c]))
    for c in range(CH):
        ocps[c].wait()


def kernel(inputs, domain_indicator, gamma, beta):
    x = inputs.reshape(-1)
    di = domain_indicator.reshape(-1)
    parts, didx = _stats_kernel(x, di)
    out = _norm_kernel(x, didx, parts,
                       gamma[:, :D].reshape(-1), beta[:, :D].reshape(-1))
    return out.reshape(B, D)
